# Initial kernel scaffold; baseline (speedup 1.0000x reference)
import functools

import jax
import jax.numpy as jnp
from jax import lax
from jax.experimental import pallas as pl
from jax.experimental.pallas import tpu as pltpu
from jax.experimental.pallas import tpu_sc as plsc

N = 10000
E = 320000
G = 256
IN_DIM = 128
H = 256
HH = 128

NSUB = 16
EB = 80
NEB = (E // NSUB) // EB
RSLAB = 640
RB = 80

_mesh = plsc.VectorSubcoreMesh(core_axis_name="c", subcore_axis_name="s")


def _fill(ref, n16, value):
    v = jnp.full((16,), value, dtype=jnp.float32)
    for k in range(n16):
        ref[pl.ds(16 * k, 16)] = v


@functools.partial(
    pl.kernel,
    out_type=[jax.ShapeDtypeStruct((N,), jnp.float32),
              jax.ShapeDtypeStruct((N,), jnp.float32)],
    mesh=_mesh,
    scratch_types=[pltpu.VMEM((EB,), jnp.int32),
                   pltpu.VMEM((EB,), jnp.float32),
                   pltpu.VMEM((EB,), jnp.float32),
                   pltpu.VMEM_SHARED((N,), jnp.float32)],
)
def _sc_degrees(src_hbm, dst_hbm, ind_out, outd_out, idxb, onesb, zb, acc):
    c = lax.axis_index("c")
    s = lax.axis_index("s")
    _fill(onesb, EB // 16, 1.0)
    _fill(zb, EB // 16, 0.0)
    base_r = s * RSLAB
    nch = jnp.where(s == NSUB - 1, (N - (NSUB - 1) * RSLAB) // RB, RSLAB // RB)

    def zloop(j, _):
        pltpu.sync_copy(zb, acc.at[pl.ds(base_r + j * RB, RB)])
        return 0
    lax.fori_loop(0, nch, zloop, 0)
    plsc.subcore_barrier()

    ebase = s * (E // NSUB)

    def eloop(j, _):
        off = ebase + j * EB

        @pl.when(c == 0)
        def _():
            pltpu.sync_copy(dst_hbm.at[pl.ds(off, EB)], idxb)

        @pl.when(c == 1)
        def _():
            pltpu.sync_copy(src_hbm.at[pl.ds(off, EB)], idxb)

        pltpu.sync_copy(onesb, acc.at[idxb], add=True)
        return 0
    lax.fori_loop(0, NEB, eloop, 0)
    plsc.subcore_barrier()

    def oloop(j, _):
        st = base_r + j * RB

        @pl.when(c == 0)
        def _():
            pltpu.sync_copy(acc.at[pl.ds(st, RB)], ind_out.at[pl.ds(st, RB)])

        @pl.when(c == 1)
        def _():
            pltpu.sync_copy(acc.at[pl.ds(st, RB)], outd_out.at[pl.ds(st, RB)])

        return 0
    lax.fori_loop(0, nch, oloop, 0)


@functools.partial(
    pl.kernel,
    out_type=[jax.ShapeDtypeStruct((N, HH), jnp.float32),
              jax.ShapeDtypeStruct((N, HH), jnp.float32)],
    mesh=_mesh,
    scratch_types=[pltpu.VMEM((EB,), jnp.int32),
                   pltpu.VMEM((EB,), jnp.int32),
                   pltpu.VMEM((EB, HH), jnp.float32),
                   pltpu.VMEM((RB, HH), jnp.float32),
                   pltpu.VMEM_SHARED((N, HH), jnp.float32),
                   pltpu.SemaphoreType.DMA],
)
def _sc_aggregate(u0, u1, src_hbm, dst_hbm, o0, o1,
                  sidx, didx, gbuf, zbuf, acc, gsem):
    c = lax.axis_index("c")
    s = lax.axis_index("s")
    z = jnp.zeros((16,), dtype=jnp.float32)
    for i in range(RB):
        for k in range(HH // 16):
            zbuf[i, pl.ds(16 * k, 16)] = z

    base_r = s * RSLAB
    nch = jnp.where(s == NSUB - 1, (N - (NSUB - 1) * RSLAB) // RB, RSLAB // RB)

    def zloop(j, _):
        pltpu.sync_copy(zbuf, acc.at[pl.ds(base_r + j * RB, RB)])
        return 0
    lax.fori_loop(0, nch, zloop, 0)
    plsc.subcore_barrier()

    ebase = s * (E // NSUB)

    def eloop(j, _):
        off = ebase + j * EB
        pltpu.sync_copy(src_hbm.at[pl.ds(off, EB)], sidx)
        pltpu.sync_copy(dst_hbm.at[pl.ds(off, EB)], didx)

        @pl.when(c == 0)
        def _():
            pltpu.async_copy(u0.at[sidx], gbuf, gsem).wait()

        @pl.when(c == 1)
        def _():
            pltpu.async_copy(u1.at[sidx], gbuf, gsem).wait()

        pltpu.sync_copy(gbuf, acc.at[didx], add=True)
        return 0
    lax.fori_loop(0, NEB, eloop, 0)
    plsc.subcore_barrier()

    def oloop(j, _):
        st = base_r + j * RB

        @pl.when(c == 0)
        def _():
            pltpu.sync_copy(acc.at[pl.ds(st, RB)], o0.at[pl.ds(st, RB)])

        @pl.when(c == 1)
        def _():
            pltpu.sync_copy(acc.at[pl.ds(st, RB)], o1.at[pl.ds(st, RB)])

        return 0
    lax.fori_loop(0, nch, oloop, 0)


RBLK = 1000
NBLK = N // RBLK


def _t0_body(x_ref, deg_ref, w_ref, u0_ref, u1_ref, inv_ref):
    deg = deg_ref[...]
    inv = lax.rsqrt(jnp.maximum(deg, 1.0))
    inv_ref[...] = inv
    xw = x_ref[...] * inv[1][:, None]
    w = w_ref[...]
    u0_ref[...] = jnp.dot(xw, w[:, :HH], preferred_element_type=jnp.float32)
    u1_ref[...] = jnp.dot(xw, w[:, HH:], preferred_element_type=jnp.float32)


_t0 = pl.pallas_call(
    _t0_body,
    grid=(NBLK,),
    in_specs=[pl.BlockSpec((RBLK, IN_DIM), lambda i: (i, 0)),
              pl.BlockSpec((2, RBLK), lambda i: (0, i)),
              pl.BlockSpec((IN_DIM, H), lambda i: (0, 0))],
    out_specs=[pl.BlockSpec((RBLK, HH), lambda i: (i, 0)),
               pl.BlockSpec((RBLK, HH), lambda i: (i, 0)),
               pl.BlockSpec((2, RBLK), lambda i: (0, i))],
    out_shape=[jax.ShapeDtypeStruct((N, HH), jnp.float32),
               jax.ShapeDtypeStruct((N, HH), jnp.float32),
               jax.ShapeDtypeStruct((2, N), jnp.float32)],
)


def _tmid_body(a0_ref, a1_ref, inv_ref, b_ref, w_ref, u0_ref, u1_ref,
               *, relu_in):
    inv = inv_ref[...]
    inv_in = inv[0][:, None]
    inv_out = inv[1][:, None]
    b = b_ref[...]
    act0 = a0_ref[...] * inv_in + b[:, :HH]
    act1 = a1_ref[...] * inv_in + b[:, HH:]
    if relu_in:
        act0 = jnp.maximum(act0, 0.0)
        act1 = jnp.maximum(act1, 0.0)
    act0 = act0 * inv_out
    act1 = act1 * inv_out
    w = w_ref[...]
    u0_ref[...] = (jnp.dot(act0, w[:HH, :HH], preferred_element_type=jnp.float32)
                   + jnp.dot(act1, w[HH:, :HH], preferred_element_type=jnp.float32))
    u1_ref[...] = (jnp.dot(act0, w[:HH, HH:], preferred_element_type=jnp.float32)
                   + jnp.dot(act1, w[HH:, HH:], preferred_element_type=jnp.float32))


def _make_tmid(relu_in):
    return pl.pallas_call(
        functools.partial(_tmid_body, relu_in=relu_in),
        grid=(NBLK,),
        in_specs=[pl.BlockSpec((RBLK, HH), lambda i: (i, 0)),
                  pl.BlockSpec((RBLK, HH), lambda i: (i, 0)),
                  pl.BlockSpec((2, RBLK), lambda i: (0, i)),
                  pl.BlockSpec((1, H), lambda i: (0, 0)),
                  pl.BlockSpec((H, H), lambda i: (0, 0))],
        out_specs=[pl.BlockSpec((RBLK, HH), lambda i: (i, 0)),
                   pl.BlockSpec((RBLK, HH), lambda i: (i, 0))],
        out_shape=[jax.ShapeDtypeStruct((N, HH), jnp.float32),
                   jax.ShapeDtypeStruct((N, HH), jnp.float32)],
    )


_tmid_first = _make_tmid(False)
_tmid_rest = _make_tmid(True)


def _leaky(x):
    return jnp.where(x >= 0, x, 0.01 * x)


def _tf_body(a0_ref, a1_ref, inv_ref, b_ref, gid_ref,
             w1_ref, b1_ref, w2_ref, b2_ref, w3_ref, b3_ref, out_ref,
             p0_ref, p1_ref, cnt_ref):
    i = pl.program_id(0)

    @pl.when(i == 0)
    def _():
        p0_ref[...] = jnp.zeros_like(p0_ref)
        p1_ref[...] = jnp.zeros_like(p1_ref)
        cnt_ref[...] = jnp.zeros_like(cnt_ref)

    inv = inv_ref[...]
    inv_in = inv[0][:, None]
    b = b_ref[...]
    h0 = jnp.maximum(a0_ref[...] * inv_in + b[:, :HH], 0.0)
    h1 = jnp.maximum(a1_ref[...] * inv_in + b[:, HH:], 0.0)
    gid = gid_ref[...]
    rows = lax.broadcasted_iota(jnp.int32, (G, RBLK), 0)
    m = (rows == gid).astype(jnp.float32)
    p0_ref[...] += jnp.dot(m, h0, preferred_element_type=jnp.float32)
    p1_ref[...] += jnp.dot(m, h1, preferred_element_type=jnp.float32)
    cnt_ref[...] += jnp.broadcast_to(jnp.sum(m, axis=1)[:, None], (G, 128))

    @pl.when(i == NBLK - 1)
    def _():
        cnt = jnp.maximum(cnt_ref[...][:, :1], 1.0)
        nm = jnp.concatenate([p0_ref[...], p1_ref[...]], axis=1) / cnt
        z = _leaky(jnp.dot(nm, w1_ref[...], preferred_element_type=jnp.float32)
                   + b1_ref[...])
        z = _leaky(jnp.dot(z, w2_ref[...], preferred_element_type=jnp.float32)
                   + b2_ref[...])
        o = (jnp.dot(z, w3_ref[...], preferred_element_type=jnp.float32)
             + b3_ref[...])
        out_ref[...] = jnp.broadcast_to(o, (G, 128))


_tf = pl.pallas_call(
    _tf_body,
    grid=(NBLK,),
    in_specs=[pl.BlockSpec((RBLK, HH), lambda i: (i, 0)),
              pl.BlockSpec((RBLK, HH), lambda i: (i, 0)),
              pl.BlockSpec((2, RBLK), lambda i: (0, i)),
              pl.BlockSpec((1, H), lambda i: (0, 0)),
              pl.BlockSpec((1, RBLK), lambda i: (0, i)),
              pl.BlockSpec((H, 1024), lambda i: (0, 0)),
              pl.BlockSpec((1, 1024), lambda i: (0, 0)),
              pl.BlockSpec((1024, 512), lambda i: (0, 0)),
              pl.BlockSpec((1, 512), lambda i: (0, 0)),
              pl.BlockSpec((512, 1), lambda i: (0, 0)),
              pl.BlockSpec((1, 1), lambda i: (0, 0))],
    out_specs=pl.BlockSpec((G, 128), lambda i: (0, 0)),
    out_shape=jax.ShapeDtypeStruct((G, 128), jnp.float32),
    scratch_shapes=[pltpu.VMEM((G, HH), jnp.float32),
                    pltpu.VMEM((G, HH), jnp.float32),
                    pltpu.VMEM((G, 128), jnp.float32)],
)


def kernel(node_features, edge_index, graph_ids,
           W0, b0, Wg, bg, W1, b1, W2, b2, W3, b3):
    src = edge_index[0]
    dst = edge_index[1]
    in_deg, out_deg = _sc_degrees(src, dst)
    deg = jnp.stack([in_deg, out_deg])

    u0, u1, inv = _t0(node_features, deg, W0)
    for j in range(11):
        a0, a1 = _sc_aggregate(u0, u1, src, dst)
        if j < 10:
            bj = (b0 if j == 0 else bg[j - 1]).reshape(1, H)
            tm = _tmid_first if j == 0 else _tmid_rest
            u0, u1 = tm(a0, a1, inv, bj, Wg[j])
        else:
            out2 = _tf(a0, a1, inv, bg[9].reshape(1, H),
                       graph_ids.reshape(1, N),
                       W1, b1.reshape(1, 1024), W2, b2.reshape(1, 512),
                       W3, b3.reshape(1, 1))
    return out2[:, 0]


# speedup vs baseline: 3.1807x; 3.1807x over previous
import functools

import jax
import jax.numpy as jnp
from jax import lax
from jax.experimental import pallas as pl
from jax.experimental.pallas import tpu as pltpu
from jax.experimental.pallas import tpu_sc as plsc

N = 10000
E = 320000
G = 256
IN_DIM = 128
H = 256
HH = 128

NSUB = 16
EB = 80
NEB = (E // NSUB) // EB
RSLAB = 640
RB = 80

def _fill(ref, n16, value):
    v = jnp.full((16,), value, dtype=jnp.float32)
    for k in range(n16):
        ref[pl.ds(16 * k, 16)] = v


def _sc_degrees_body(src_hbm, dst_hbm, ind_out, outd_out, idxb, onesb, zb, acc):
    c = lax.axis_index("c")
    s = lax.axis_index("s")
    _fill(onesb, EB // 16, 1.0)
    _fill(zb, EB // 16, 0.0)
    base_r = s * RSLAB
    nch = jnp.where(s == NSUB - 1, (N - (NSUB - 1) * RSLAB) // RB, RSLAB // RB)

    def zloop(j, _):
        pltpu.sync_copy(zb, acc.at[pl.ds(base_r + j * RB, RB)])
        return 0
    lax.fori_loop(0, nch, zloop, 0)
    plsc.subcore_barrier()

    ebase = s * (E // NSUB)

    def eloop(j, _):
        off = ebase + j * EB

        @pl.when(c == 0)
        def _():
            pltpu.sync_copy(dst_hbm.at[pl.ds(off, EB)], idxb)

        @pl.when(c == 1)
        def _():
            pltpu.sync_copy(src_hbm.at[pl.ds(off, EB)], idxb)

        pltpu.sync_copy(onesb, acc.at[idxb], add=True)
        return 0
    lax.fori_loop(0, NEB, eloop, 0)
    plsc.subcore_barrier()

    def oloop(j, _):
        st = base_r + j * RB
        pltpu.sync_copy(acc.at[pl.ds(st, RB)], zb)

        @pl.when(c == 0)
        def _():
            pltpu.sync_copy(zb, ind_out.at[pl.ds(st, RB)])

        @pl.when(c == 1)
        def _():
            pltpu.sync_copy(zb, outd_out.at[pl.ds(st, RB)])

        return 0
    lax.fori_loop(0, nch, oloop, 0)


def _sc_aggregate_body(u0, u1, src_hbm, dst_hbm, o0, o1,
                       sidx, didx, gbuf, zbuf, acc, gsem):
    c = lax.axis_index("c")
    s = lax.axis_index("s")
    z = jnp.zeros((16,), dtype=jnp.float32)
    for i in range(RB):
        for k in range(HH // 16):
            zbuf[i, pl.ds(16 * k, 16)] = z

    base_r = s * RSLAB
    nch = jnp.where(s == NSUB - 1, (N - (NSUB - 1) * RSLAB) // RB, RSLAB // RB)

    def zloop(j, _):
        pltpu.sync_copy(zbuf, acc.at[pl.ds(base_r + j * RB, RB)])
        return 0
    lax.fori_loop(0, nch, zloop, 0)
    plsc.subcore_barrier()

    ebase = s * (E // NSUB)

    def eloop(j, _):
        off = ebase + j * EB
        pltpu.sync_copy(src_hbm.at[pl.ds(off, EB)], sidx)
        pltpu.sync_copy(dst_hbm.at[pl.ds(off, EB)], didx)

        @pl.when(c == 0)
        def _():
            pltpu.async_copy(u0.at[sidx], gbuf, gsem).wait()

        @pl.when(c == 1)
        def _():
            pltpu.async_copy(u1.at[sidx], gbuf, gsem).wait()

        pltpu.sync_copy(gbuf, acc.at[didx], add=True)
        return 0
    lax.fori_loop(0, NEB, eloop, 0)
    plsc.subcore_barrier()

    def oloop(j, _):
        st = base_r + j * RB
        pltpu.sync_copy(acc.at[pl.ds(st, RB)], zbuf)

        @pl.when(c == 0)
        def _():
            pltpu.sync_copy(zbuf, o0.at[pl.ds(st, RB)])

        @pl.when(c == 1)
        def _():
            pltpu.sync_copy(zbuf, o1.at[pl.ds(st, RB)])

        return 0
    lax.fori_loop(0, nch, oloop, 0)


@functools.cache
def _get_sc_kernels():
    mesh = plsc.VectorSubcoreMesh(core_axis_name="c", subcore_axis_name="s")
    sc_degrees = pl.kernel(
        _sc_degrees_body,
        out_type=[jax.ShapeDtypeStruct((N,), jnp.float32),
                  jax.ShapeDtypeStruct((N,), jnp.float32)],
        mesh=mesh,
        scratch_types=[pltpu.VMEM((EB,), jnp.int32),
                       pltpu.VMEM((EB,), jnp.float32),
                       pltpu.VMEM((EB,), jnp.float32),
                       pltpu.VMEM_SHARED((N,), jnp.float32)],
    )
    sc_aggregate = pl.kernel(
        _sc_aggregate_body,
        out_type=[jax.ShapeDtypeStruct((N, HH), jnp.float32),
                  jax.ShapeDtypeStruct((N, HH), jnp.float32)],
        mesh=mesh,
        scratch_types=[pltpu.VMEM((EB,), jnp.int32),
                       pltpu.VMEM((EB,), jnp.int32),
                       pltpu.VMEM((EB, HH), jnp.float32),
                       pltpu.VMEM((RB, HH), jnp.float32),
                       pltpu.VMEM_SHARED((N, HH), jnp.float32),
                       pltpu.SemaphoreType.DMA],
    )
    return sc_degrees, sc_aggregate


def _sc_degrees(src, dst):
    return _get_sc_kernels()[0](src, dst)


def _sc_aggregate(u0, u1, src, dst):
    return _get_sc_kernels()[1](u0, u1, src, dst)


RBLK = 1000
NBLK = N // RBLK


def _t0_body(x_ref, deg_ref, w_ref, u0_ref, u1_ref, inv_ref):
    deg = deg_ref[...]
    inv = lax.rsqrt(jnp.maximum(deg, 1.0))
    inv_ref[...] = inv
    xw = x_ref[...] * inv[:, 1:2]
    w = w_ref[...]
    u0_ref[...] = jnp.dot(xw, w[:, :HH], preferred_element_type=jnp.float32)
    u1_ref[...] = jnp.dot(xw, w[:, HH:], preferred_element_type=jnp.float32)


def _build_t0(interpret=False):
    return pl.pallas_call(
        _t0_body,
        grid=(NBLK,),
        in_specs=[pl.BlockSpec((RBLK, IN_DIM), lambda i: (i, 0)),
                  pl.BlockSpec((RBLK, 2), lambda i: (i, 0)),
                  pl.BlockSpec((IN_DIM, H), lambda i: (0, 0))],
        out_specs=[pl.BlockSpec((RBLK, HH), lambda i: (i, 0)),
                   pl.BlockSpec((RBLK, HH), lambda i: (i, 0)),
                   pl.BlockSpec((RBLK, 2), lambda i: (i, 0))],
        out_shape=[jax.ShapeDtypeStruct((N, HH), jnp.float32),
                   jax.ShapeDtypeStruct((N, HH), jnp.float32),
                   jax.ShapeDtypeStruct((N, 2), jnp.float32)],
        interpret=interpret,
    )


_t0 = _build_t0()


def _tmid_body(a0_ref, a1_ref, inv_ref, b_ref, w_ref, u0_ref, u1_ref,
               *, relu_in):
    inv = inv_ref[...]
    inv_in = inv[:, 0:1]
    inv_out = inv[:, 1:2]
    b = b_ref[...]
    act0 = a0_ref[...] * inv_in + b[:, :HH]
    act1 = a1_ref[...] * inv_in + b[:, HH:]
    if relu_in:
        act0 = jnp.maximum(act0, 0.0)
        act1 = jnp.maximum(act1, 0.0)
    act0 = act0 * inv_out
    act1 = act1 * inv_out
    w = w_ref[...]
    u0_ref[...] = (jnp.dot(act0, w[:HH, :HH], preferred_element_type=jnp.float32)
                   + jnp.dot(act1, w[HH:, :HH], preferred_element_type=jnp.float32))
    u1_ref[...] = (jnp.dot(act0, w[:HH, HH:], preferred_element_type=jnp.float32)
                   + jnp.dot(act1, w[HH:, HH:], preferred_element_type=jnp.float32))


def _make_tmid(relu_in, interpret=False):
    return pl.pallas_call(
        functools.partial(_tmid_body, relu_in=relu_in),
        interpret=interpret,
        grid=(NBLK,),
        in_specs=[pl.BlockSpec((RBLK, HH), lambda i: (i, 0)),
                  pl.BlockSpec((RBLK, HH), lambda i: (i, 0)),
                  pl.BlockSpec((RBLK, 2), lambda i: (i, 0)),
                  pl.BlockSpec((1, H), lambda i: (0, 0)),
                  pl.BlockSpec((H, H), lambda i: (0, 0))],
        out_specs=[pl.BlockSpec((RBLK, HH), lambda i: (i, 0)),
                   pl.BlockSpec((RBLK, HH), lambda i: (i, 0))],
        out_shape=[jax.ShapeDtypeStruct((N, HH), jnp.float32),
                   jax.ShapeDtypeStruct((N, HH), jnp.float32)],
    )


_tmid_first = _make_tmid(False)
_tmid_rest = _make_tmid(True)


def _leaky(x):
    return jnp.where(x >= 0, x, 0.01 * x)


def _tf_body(a0_ref, a1_ref, inv_ref, b_ref, gid_ref,
             w1_ref, b1_ref, w2_ref, b2_ref, w3_ref, b3_ref, out_ref,
             p0_ref, p1_ref, cnt_ref):
    i = pl.program_id(0)

    @pl.when(i == 0)
    def _():
        p0_ref[...] = jnp.zeros_like(p0_ref)
        p1_ref[...] = jnp.zeros_like(p1_ref)
        cnt_ref[...] = jnp.zeros_like(cnt_ref)

    inv = inv_ref[...]
    inv_in = inv[:, 0:1]
    b = b_ref[...]
    h0 = jnp.maximum(a0_ref[...] * inv_in + b[:, :HH], 0.0)
    h1 = jnp.maximum(a1_ref[...] * inv_in + b[:, HH:], 0.0)
    gid = gid_ref[...][:, 0]
    rows = lax.broadcasted_iota(jnp.int32, (G, RBLK), 0)
    m = (rows == gid[None, :]).astype(jnp.float32)
    p0_ref[...] += jnp.dot(m, h0, preferred_element_type=jnp.float32)
    p1_ref[...] += jnp.dot(m, h1, preferred_element_type=jnp.float32)
    cnt_ref[...] += jnp.broadcast_to(jnp.sum(m, axis=1)[:, None], (G, 128))

    @pl.when(i == NBLK - 1)
    def _():
        cnt = jnp.maximum(cnt_ref[...][:, :1], 1.0)
        nm = jnp.concatenate([p0_ref[...], p1_ref[...]], axis=1) / cnt
        z = _leaky(jnp.dot(nm, w1_ref[...], preferred_element_type=jnp.float32)
                   + b1_ref[...])
        z = _leaky(jnp.dot(z, w2_ref[...], preferred_element_type=jnp.float32)
                   + b2_ref[...])
        o = (jnp.dot(z, w3_ref[...], preferred_element_type=jnp.float32)
             + b3_ref[...])
        out_ref[...] = jnp.broadcast_to(o, (G, 128))


def _build_tf(interpret=False):
    return pl.pallas_call(
    _tf_body,
    interpret=interpret,
    grid=(NBLK,),
    in_specs=[pl.BlockSpec((RBLK, HH), lambda i: (i, 0)),
              pl.BlockSpec((RBLK, HH), lambda i: (i, 0)),
              pl.BlockSpec((RBLK, 2), lambda i: (i, 0)),
              pl.BlockSpec((1, H), lambda i: (0, 0)),
              pl.BlockSpec((RBLK, 1), lambda i: (i, 0)),
              pl.BlockSpec((H, 1024), lambda i: (0, 0)),
              pl.BlockSpec((1, 1024), lambda i: (0, 0)),
              pl.BlockSpec((1024, 512), lambda i: (0, 0)),
              pl.BlockSpec((1, 512), lambda i: (0, 0)),
              pl.BlockSpec((512, 1), lambda i: (0, 0)),
              pl.BlockSpec((1, 1), lambda i: (0, 0))],
    out_specs=pl.BlockSpec((G, 128), lambda i: (0, 0)),
    out_shape=jax.ShapeDtypeStruct((G, 128), jnp.float32),
    scratch_shapes=[pltpu.VMEM((G, HH), jnp.float32),
                    pltpu.VMEM((G, HH), jnp.float32),
                    pltpu.VMEM((G, 128), jnp.float32)],
    )


_tf = _build_tf()


def kernel(node_features, edge_index, graph_ids,
           W0, b0, Wg, bg, W1, b1, W2, b2, W3, b3):
    src = edge_index[0]
    dst = edge_index[1]
    in_deg, out_deg = _sc_degrees(src, dst)
    deg = jnp.stack([in_deg, out_deg], axis=1)

    u0, u1, inv = _t0(node_features, deg, W0)
    for j in range(11):
        a0, a1 = _sc_aggregate(u0, u1, src, dst)
        if j < 10:
            bj = (b0 if j == 0 else bg[j - 1]).reshape(1, H)
            tm = _tmid_first if j == 0 else _tmid_rest
            u0, u1 = tm(a0, a1, inv, bj, Wg[j])
        else:
            out2 = _tf(a0, a1, inv, bg[9].reshape(1, H),
                       graph_ids.reshape(N, 1),
                       W1, b1.reshape(1, 1024), W2, b2.reshape(1, 512),
                       W3, b3.reshape(1, 1))
    return out2[:, 0]


# speedup vs baseline: 7.7366x; 2.4323x over previous
import functools

import jax
import jax.numpy as jnp
from jax import lax
from jax.experimental import pallas as pl
from jax.experimental.pallas import tpu as pltpu
from jax.experimental.pallas import tpu_sc as plsc

N = 10000
E = 320000
G = 256
IN_DIM = 128
H = 256
HH = 128

NSUB = 16
EB = 80
NEB = (E // NSUB) // EB
RSLAB = 640
RB = 80

def _fill(ref, n16, value):
    v = jnp.full((16,), value, dtype=jnp.float32)
    for k in range(n16):
        ref[pl.ds(16 * k, 16)] = v


def _sc_degrees_body(src_hbm, dst_hbm, ind_out, outd_out, idxb, onesb, zb, acc):
    c = lax.axis_index("c")
    s = lax.axis_index("s")
    _fill(onesb, EB // 16, 1.0)
    _fill(zb, EB // 16, 0.0)
    base_r = s * RSLAB
    nch = jnp.where(s == NSUB - 1, (N - (NSUB - 1) * RSLAB) // RB, RSLAB // RB)

    def zloop(j, _):
        pltpu.sync_copy(zb, acc.at[pl.ds(base_r + j * RB, RB)])
        return 0
    lax.fori_loop(0, nch, zloop, 0)
    plsc.subcore_barrier()

    ebase = s * (E // NSUB)

    def eloop(j, _):
        off = ebase + j * EB

        @pl.when(c == 0)
        def _():
            pltpu.sync_copy(dst_hbm.at[pl.ds(off, EB)], idxb)

        @pl.when(c == 1)
        def _():
            pltpu.sync_copy(src_hbm.at[pl.ds(off, EB)], idxb)

        pltpu.sync_copy(onesb, acc.at[idxb], add=True)
        return 0
    lax.fori_loop(0, NEB, eloop, 0)
    plsc.subcore_barrier()

    def oloop(j, _):
        st = base_r + j * RB
        pltpu.sync_copy(acc.at[pl.ds(st, RB)], zb)

        @pl.when(c == 0)
        def _():
            pltpu.sync_copy(zb, ind_out.at[pl.ds(st, RB)])

        @pl.when(c == 1)
        def _():
            pltpu.sync_copy(zb, outd_out.at[pl.ds(st, RB)])

        return 0
    lax.fori_loop(0, nch, oloop, 0)


SEB = 80
SNB = (E // NSUB) // SEB
NSTG = 5
SCH = SNB // NSTG
ZRB = 40


def _sc_aggregate_body(u0, u1, src_hbm, dst_hbm, o0, o1,
                       sidx, didx, gbuf0, gbuf1, zbuf, acc,
                       gsem0, gsem1):
    c = lax.axis_index("c")
    s = lax.axis_index("s")
    z = jnp.zeros((16,), dtype=jnp.float32)
    for i in range(ZRB):
        for k in range(HH // 16):
            zbuf[i, pl.ds(16 * k, 16)] = z

    base_r = s * RSLAB
    nch = jnp.where(s == NSUB - 1, (N - (NSUB - 1) * RSLAB) // ZRB,
                    RSLAB // ZRB)

    def zloop(j, _):
        pltpu.sync_copy(zbuf, acc.at[pl.ds(base_r + j * ZRB, ZRB)])
        return 0
    lax.fori_loop(0, nch, zloop, 0)
    plsc.subcore_barrier()

    gbufs = (gbuf0, gbuf1)
    gsems = (gsem0, gsem1)

    def gather_start(j, b):
        @pl.when(c == 0)
        def _():
            pltpu.async_copy(u0.at[sidx.at[j]], gbufs[b], gsems[b])

        @pl.when(c == 1)
        def _():
            pltpu.async_copy(u1.at[sidx.at[j]], gbufs[b], gsems[b])

    def gather_wait(b):
        pltpu.make_async_copy(u0.at[pl.ds(0, SEB)], gbufs[b], gsems[b]).wait()

    def souter(stg, _):
        pltpu.sync_copy(src_hbm.at[s * NSTG + stg], sidx)
        pltpu.sync_copy(dst_hbm.at[s * NSTG + stg], didx)
        gather_start(0, 0)

        def eloop(k, _):
            for bb in range(2):
                j = 2 * k + bb

                @pl.when(j + 1 < SCH)
                def _():
                    gather_start(j + 1, 1 - bb)

                gather_wait(bb)
                pltpu.sync_copy(gbufs[bb], acc.at[didx.at[j]], add=True)
            return 0
        lax.fori_loop(0, SCH // 2, eloop, 0)
        return 0
    lax.fori_loop(0, NSTG, souter, 0)
    plsc.subcore_barrier()

    def oloop(j, _):
        st = base_r + j * ZRB
        pltpu.sync_copy(acc.at[pl.ds(st, ZRB)], zbuf)

        @pl.when(c == 0)
        def _():
            pltpu.sync_copy(zbuf, o0.at[pl.ds(st, ZRB)])

        @pl.when(c == 1)
        def _():
            pltpu.sync_copy(zbuf, o1.at[pl.ds(st, ZRB)])

        return 0
    lax.fori_loop(0, nch, oloop, 0)


@functools.cache
def _get_sc_kernels():
    mesh = plsc.VectorSubcoreMesh(core_axis_name="c", subcore_axis_name="s")
    sc_degrees = pl.kernel(
        _sc_degrees_body,
        out_type=[jax.ShapeDtypeStruct((N,), jnp.float32),
                  jax.ShapeDtypeStruct((N,), jnp.float32)],
        mesh=mesh,
        scratch_types=[pltpu.VMEM((EB,), jnp.int32),
                       pltpu.VMEM((EB,), jnp.float32),
                       pltpu.VMEM((EB,), jnp.float32),
                       pltpu.VMEM_SHARED((N,), jnp.float32)],
    )
    sc_aggregate = pl.kernel(
        _sc_aggregate_body,
        out_type=[jax.ShapeDtypeStruct((N, HH), jnp.float32),
                  jax.ShapeDtypeStruct((N, HH), jnp.float32)],
        mesh=mesh,
        scratch_types=[pltpu.VMEM((SCH, SEB), jnp.int32),
                       pltpu.VMEM((SCH, SEB), jnp.int32),
                       pltpu.VMEM((SEB, HH), jnp.float32),
                       pltpu.VMEM((SEB, HH), jnp.float32),
                       pltpu.VMEM((ZRB, HH), jnp.float32),
                       pltpu.VMEM_SHARED((N, HH), jnp.float32),
                       pltpu.SemaphoreType.DMA,
                       pltpu.SemaphoreType.DMA],
    )
    return sc_degrees, sc_aggregate


def _sc_degrees(src, dst):
    return _get_sc_kernels()[0](src, dst)


def _sc_aggregate(u0, u1, src2, dst2):
    return _get_sc_kernels()[1](u0, u1, src2, dst2)


RBLK = 1000
NBLK = N // RBLK


def _t0_body(x_ref, inv_ref, w_ref, u0_ref, u1_ref):
    inv = inv_ref[...]
    xw = x_ref[...] * inv[:, 1:2]
    u = jnp.dot(xw, w_ref[...], preferred_element_type=jnp.float32)
    u0_ref[...] = u[:, :HH]
    u1_ref[...] = u[:, HH:]


def _build_t0(interpret=False):
    return pl.pallas_call(
        _t0_body,
        grid=(NBLK,),
        in_specs=[pl.BlockSpec((RBLK, IN_DIM), lambda i: (i, 0)),
                  pl.BlockSpec((RBLK, 2), lambda i: (i, 0)),
                  pl.BlockSpec((IN_DIM, H), lambda i: (0, 0))],
        out_specs=[pl.BlockSpec((RBLK, HH), lambda i: (i, 0)),
                   pl.BlockSpec((RBLK, HH), lambda i: (i, 0))],
        out_shape=[jax.ShapeDtypeStruct((N, HH), jnp.float32),
                   jax.ShapeDtypeStruct((N, HH), jnp.float32)],
        interpret=interpret,
    )


_t0 = _build_t0()


def _tmid_body(a0_ref, a1_ref, inv_ref, b_ref, w_ref, u0_ref, u1_ref,
               *, relu_in):
    inv = inv_ref[...]
    inv_in = inv[:, 0:1]
    inv_out = inv[:, 1:2]
    b = b_ref[...]
    act0 = a0_ref[...] * inv_in + b[:, :HH]
    act1 = a1_ref[...] * inv_in + b[:, HH:]
    if relu_in:
        act0 = jnp.maximum(act0, 0.0)
        act1 = jnp.maximum(act1, 0.0)
    act = jnp.concatenate([act0 * inv_out, act1 * inv_out], axis=1)
    u = jnp.dot(act, w_ref[...], preferred_element_type=jnp.float32)
    u0_ref[...] = u[:, :HH]
    u1_ref[...] = u[:, HH:]


def _make_tmid(relu_in, interpret=False):
    return pl.pallas_call(
        functools.partial(_tmid_body, relu_in=relu_in),
        interpret=interpret,
        grid=(NBLK,),
        in_specs=[pl.BlockSpec((RBLK, HH), lambda i: (i, 0)),
                  pl.BlockSpec((RBLK, HH), lambda i: (i, 0)),
                  pl.BlockSpec((RBLK, 2), lambda i: (i, 0)),
                  pl.BlockSpec((1, H), lambda i: (0, 0)),
                  pl.BlockSpec((H, H), lambda i: (0, 0))],
        out_specs=[pl.BlockSpec((RBLK, HH), lambda i: (i, 0)),
                   pl.BlockSpec((RBLK, HH), lambda i: (i, 0))],
        out_shape=[jax.ShapeDtypeStruct((N, HH), jnp.float32),
                   jax.ShapeDtypeStruct((N, HH), jnp.float32)],
    )


_tmid_first = _make_tmid(False)
_tmid_rest = _make_tmid(True)


def _leaky(x):
    return jnp.where(x >= 0, x, 0.01 * x)


def _tf_body(a0_ref, a1_ref, inv_ref, b_ref, gid_ref,
             w1_ref, b1_ref, w2_ref, b2_ref, w3_ref, b3_ref, out_ref,
             p0_ref, p1_ref, cnt_ref):
    i = pl.program_id(0)

    @pl.when(i == 0)
    def _():
        p0_ref[...] = jnp.zeros_like(p0_ref)
        p1_ref[...] = jnp.zeros_like(p1_ref)
        cnt_ref[...] = jnp.zeros_like(cnt_ref)

    inv = inv_ref[...]
    inv_in = inv[:, 0:1]
    b = b_ref[...]
    h0 = jnp.maximum(a0_ref[...] * inv_in + b[:, :HH], 0.0)
    h1 = jnp.maximum(a1_ref[...] * inv_in + b[:, HH:], 0.0)
    gid = gid_ref[...][:, 0]
    rows = lax.broadcasted_iota(jnp.int32, (G, RBLK), 0)
    m = (rows == gid[None, :]).astype(jnp.float32)
    p0_ref[...] += jnp.dot(m, h0, preferred_element_type=jnp.float32,
                           precision=lax.Precision.HIGHEST)
    p1_ref[...] += jnp.dot(m, h1, preferred_element_type=jnp.float32,
                           precision=lax.Precision.HIGHEST)
    cnt_ref[...] += jnp.broadcast_to(jnp.sum(m, axis=1)[:, None], (G, 128))

    @pl.when(i == NBLK - 1)
    def _():
        cnt = jnp.maximum(cnt_ref[...][:, :1], 1.0)
        nm = jnp.concatenate([p0_ref[...], p1_ref[...]], axis=1) / cnt
        z = _leaky(jnp.dot(nm, w1_ref[...], preferred_element_type=jnp.float32)
                   + b1_ref[...])
        z = _leaky(jnp.dot(z, w2_ref[...], preferred_element_type=jnp.float32)
                   + b2_ref[...])
        o = (jnp.dot(z, w3_ref[...], preferred_element_type=jnp.float32)
             + b3_ref[...])
        out_ref[...] = jnp.broadcast_to(o, (G, 128))


def _build_tf(interpret=False):
    return pl.pallas_call(
    _tf_body,
    interpret=interpret,
    grid=(NBLK,),
    in_specs=[pl.BlockSpec((RBLK, HH), lambda i: (i, 0)),
              pl.BlockSpec((RBLK, HH), lambda i: (i, 0)),
              pl.BlockSpec((RBLK, 2), lambda i: (i, 0)),
              pl.BlockSpec((1, H), lambda i: (0, 0)),
              pl.BlockSpec((RBLK, 1), lambda i: (i, 0)),
              pl.BlockSpec((H, 1024), lambda i: (0, 0)),
              pl.BlockSpec((1, 1024), lambda i: (0, 0)),
              pl.BlockSpec((1024, 512), lambda i: (0, 0)),
              pl.BlockSpec((1, 512), lambda i: (0, 0)),
              pl.BlockSpec((512, 1), lambda i: (0, 0)),
              pl.BlockSpec((1, 1), lambda i: (0, 0))],
    out_specs=pl.BlockSpec((G, 128), lambda i: (0, 0)),
    out_shape=jax.ShapeDtypeStruct((G, 128), jnp.float32),
    scratch_shapes=[pltpu.VMEM((G, HH), jnp.float32),
                    pltpu.VMEM((G, HH), jnp.float32),
                    pltpu.VMEM((G, 128), jnp.float32)],
    )


_tf = _build_tf()


def kernel(node_features, edge_index, graph_ids,
           W0, b0, Wg, bg, W1, b1, W2, b2, W3, b3):
    src = edge_index[0]
    dst = edge_index[1]
    src2 = src.reshape(NSUB * NSTG, SCH, SEB)
    dst2 = dst.reshape(NSUB * NSTG, SCH, SEB)
    in_deg, out_deg = _sc_degrees(src, dst)
    inv = jnp.stack([jnp.clip(in_deg, 1.0) ** -0.5,
                     jnp.clip(out_deg, 1.0) ** -0.5], axis=1)

    u0, u1 = _t0(node_features, inv, W0)
    for j in range(11):
        a0, a1 = _sc_aggregate(u0, u1, src2, dst2)
        if j < 10:
            bj = (b0 if j == 0 else bg[j - 1]).reshape(1, H)
            tm = _tmid_first if j == 0 else _tmid_rest
            u0, u1 = tm(a0, a1, inv, bj, Wg[j])
        else:
            out2 = _tf(a0, a1, inv, bg[9].reshape(1, H),
                       graph_ids.reshape(N, 1),
                       W1, b1.reshape(1, 1024), W2, b2.reshape(1, 512),
                       W3, b3.reshape(1, 1))
    return out2[:, 0]
